# Initial kernel scaffold; baseline (speedup 1.0000x reference)
import jax
import jax.numpy as jnp
from jax.experimental import pallas as pl


def kernel(box_features, phrase_embed, target_id, W1_sim, b1_sim, W2_sim, b2_sim, W1_reg, b1_reg, W2_reg, b2_reg, W1_sim_topN, b1_sim_topN, W2_sim_topN, b2_sim_topN, W1_reg_topN, b1_reg_topN, W2_reg_topN, b2_reg_topN):
    raise NotImplementedError("write your pallas kernel here")


# speedup vs baseline: 1.7223x; 1.7223x over previous
import functools

import jax
import jax.numpy as jnp
from jax import lax
from jax.experimental import pallas as pl
from jax.experimental.pallas import tpu as pltpu

_B, _P, _N, _K = 4, 25, 256, 8
_D_REC, _D_PHR = 128, 768
_HID = 256
_NEG = -1e9


def _leaky(x):
    return jnp.where(x > 0, x, 0.01 * x)


def _body(tid_ref, boxT_ref, phr_ref,
          W1s_bT, W1s_pT, b1s, w2sT, b2s,
          W1st_bT, W1st_pT, b1st, w2stT, b2st,
          W1rt_bT, W1rt_pT, b1rt, W2rtT, b2rt,
          sim_out, det_out, reg_out):
    f32 = jnp.float32
    boxT = boxT_ref[0]
    phr_col = phr_ref[0]

    proj_s = jnp.dot(W1s_pT[...], phr_col, preferred_element_type=f32)
    hT = jnp.dot(W1s_bT[...], boxT, preferred_element_type=f32)
    hT = _leaky(hT + proj_s + b1s[...])
    s_row = jnp.dot(w2sT[...], hT, preferred_element_type=f32) + b2s[...]
    sim_out[0] = s_row

    lane_n = lax.broadcasted_iota(jnp.int32, (1, _N), 1)
    lane_k = lax.broadcasted_iota(jnp.int32, (1, _K), 1)
    sub_k = lax.broadcasted_iota(jnp.int32, (_K, _N), 0)
    lane_n2 = lax.broadcasted_iota(jnp.int32, (_K, _N), 1)
    sub_n = lax.broadcasted_iota(jnp.int32, (_N, _K), 0)
    lane_k2 = lax.broadcasted_iota(jnp.int32, (_N, _K), 1)

    work = s_row
    scores_row = jnp.zeros((1, _K), f32)
    onehot = jnp.zeros((_K, _N), f32)
    onehotT = jnp.zeros((_N, _K), f32)
    for k in range(_K):
        m = jnp.max(work)
        idx = jnp.min(jnp.where(work == m, lane_n, _N))
        scores_row = jnp.where(lane_k == k, m, scores_row)
        onehot = jnp.where((sub_k == k) & (lane_n2 == idx), 1.0, onehot)
        onehotT = jnp.where((lane_k2 == k) & (sub_n == idx), 1.0, onehotT)
        work = jnp.where(lane_n == idx, -jnp.inf, work)

    gathT = jnp.dot(boxT, onehotT, preferred_element_type=f32)

    proj_st = jnp.dot(W1st_pT[...], phr_col, preferred_element_type=f32)
    h2T = _leaky(jnp.dot(W1st_bT[...], gathT, preferred_element_type=f32)
                 + proj_st + b1st[...])
    sim2_row = jnp.dot(w2stT[...], h2T, preferred_element_type=f32) + b2st[...]

    proj_rt = jnp.dot(W1rt_pT[...], phr_col, preferred_element_type=f32)
    h3T = _leaky(jnp.dot(W1rt_bT[...], gathT, preferred_element_type=f32)
                 + proj_rt + b1rt[...])
    regT = jnp.dot(W2rtT[...], h3T, preferred_element_type=f32) + b2rt[...]
    reg_out[0] = regT

    fused_row = sim2_row * scores_row
    det = jnp.dot(fused_row, onehot, preferred_element_type=f32)
    cov = jnp.dot(jnp.ones((1, _K), f32), onehot, preferred_element_type=f32)
    det_out[0] = jnp.where(cov > 0.5, det, _NEG)


@jax.jit
def kernel(box_features, phrase_embed, target_id,
           W1_sim, b1_sim, W2_sim, b2_sim,
           W1_reg, b1_reg, W2_reg, b2_reg,
           W1_sim_topN, b1_sim_topN, W2_sim_topN, b2_sim_topN,
           W1_reg_topN, b1_reg_topN, W2_reg_topN, b2_reg_topN):
    del W1_reg, b1_reg, W2_reg, b2_reg

    boxT = box_features.transpose(0, 2, 1)
    phr = phrase_embed.reshape(_B * _P, _D_PHR, 1)

    f32 = jnp.float32
    args = (
        boxT, phr,
        W1_sim[:_D_REC].T, W1_sim[_D_REC:].T, b1_sim.reshape(_HID, 1),
        W2_sim.T, b2_sim.reshape(1, 1),
        W1_sim_topN[:_D_REC].T, W1_sim_topN[_D_REC:].T, b1_sim_topN.reshape(_HID, 1),
        W2_sim_topN.T, b2_sim_topN.reshape(1, 1),
        W1_reg_topN[:_D_REC].T, W1_reg_topN[_D_REC:].T, b1_reg_topN.reshape(_HID, 1),
        W2_reg_topN.T, b2_reg_topN.reshape(6, 1),
    )

    def full(a):
        return pl.BlockSpec(a.shape, lambda b, tid: (0,) * a.ndim)

    in_specs = [
        pl.BlockSpec((1, _D_REC, _N), lambda b, tid: (b, 0, 0)),
        pl.BlockSpec((1, _D_PHR, 1), lambda b, tid: (b * _P + tid[b], 0, 0)),
    ] + [full(a) for a in args[2:]]

    grid_spec = pltpu.PrefetchScalarGridSpec(
        num_scalar_prefetch=1,
        grid=(_B,),
        in_specs=in_specs,
        out_specs=[
            pl.BlockSpec((1, 1, _N), lambda b, tid: (b, 0, 0)),
            pl.BlockSpec((1, 1, _N), lambda b, tid: (b, 0, 0)),
            pl.BlockSpec((1, 6, _K), lambda b, tid: (b, 0, 0)),
        ],
    )

    sim3, det3, regT3 = pl.pallas_call(
        _body,
        grid_spec=grid_spec,
        out_shape=[
            jax.ShapeDtypeStruct((_B, 1, _N), f32),
            jax.ShapeDtypeStruct((_B, 1, _N), f32),
            jax.ShapeDtypeStruct((_B, 6, _K), f32),
        ],
    )(target_id, *args)

    sim_target = sim3.reshape(_B, _N)
    det = det3.reshape(_B, _N)
    reg_target = regT3.transpose(0, 2, 1)
    return sim_target, det, reg_target


# speedup vs baseline: 5.4790x; 3.1812x over previous
import jax
import jax.numpy as jnp
from jax import lax
from jax.experimental import pallas as pl
from jax.experimental.pallas import tpu as pltpu

_B, _P, _N, _K = 4, 25, 256, 8
_D_REC, _D_PHR = 128, 768
_HID = 256
_NEG = -1e9


def _leaky(x):
    return jnp.where(x > 0, x, 0.01 * x)


def _body(tid_ref, box_ref, phr_ref,
          W1s_ref, b1s_ref, W2s_ref, b2s_ref,
          W1st_ref, b1st_ref, W2st_ref, b2st_ref,
          W1rt_ref, b1rt_ref, W2rt_ref, b2rt_ref,
          sim_out, det_out, reg_out):
    f32 = jnp.float32

    sub_b = lax.broadcasted_iota(jnp.int32, (_B, _B * _P), 0)
    lane_bp = lax.broadcasted_iota(jnp.int32, (_B, _B * _P), 1)
    sel = jnp.zeros((_B, _B * _P), f32)
    for b in range(_B):
        sel = jnp.where((sub_b == b) & (lane_bp == b * _P + tid_ref[b]), 1.0, sel)
    phrs = jnp.dot(sel, phr_ref[...], preferred_element_type=f32)

    box_all = box_ref[...]
    sub_bn = lax.broadcasted_iota(jnp.int32, (_B * _N, 1), 0)
    exp_bn = jnp.where(
        (sub_bn // _N) == lax.broadcasted_iota(jnp.int32, (_B * _N, _B), 1), 1.0, 0.0)
    pair = jnp.concatenate(
        [box_all, jnp.dot(exp_bn, phrs, preferred_element_type=f32)], axis=1)
    h = _leaky(jnp.dot(pair, W1s_ref[...], preferred_element_type=f32)
               + b1s_ref[...])
    sim_all = jnp.dot(h, W2s_ref[...], preferred_element_type=f32) + b2s_ref[...]
    sim_out[...] = sim_all

    sub_n = lax.broadcasted_iota(jnp.int32, (_N, 1), 0)
    sub_k = lax.broadcasted_iota(jnp.int32, (_K, 1), 0)
    sub_kn0 = lax.broadcasted_iota(jnp.int32, (_K, _N), 0)
    lane_kn1 = lax.broadcasted_iota(jnp.int32, (_K, _N), 1)
    sub_nk0 = lax.broadcasted_iota(jnp.int32, (_N, _K), 0)
    lane_nk1 = lax.broadcasted_iota(jnp.int32, (_N, _K), 1)

    scores = []
    onehots = []
    onehotsT = []
    for b in range(_B):
        work = sim_all[b * _N:(b + 1) * _N, :]
        sc = jnp.zeros((_K, 1), f32)
        oh = jnp.zeros((_K, _N), f32)
        ohT = jnp.zeros((_N, _K), f32)
        for k in range(_K):
            m = jnp.max(work)
            idx = jnp.min(jnp.where(work == m, sub_n, _N))
            sc = jnp.where(sub_k == k, m, sc)
            oh = jnp.where((sub_kn0 == k) & (lane_kn1 == idx), 1.0, oh)
            ohT = jnp.where((sub_nk0 == idx) & (lane_nk1 == k), 1.0, ohT)
            work = jnp.where(sub_n == idx, -jnp.inf, work)
        scores.append(sc)
        onehots.append(oh)
        onehotsT.append(ohT)

    gath = jnp.concatenate(
        [jnp.dot(onehots[b], box_all[b * _N:(b + 1) * _N, :],
                 preferred_element_type=f32) for b in range(_B)], axis=0)
    scores_all = jnp.concatenate(scores, axis=0)

    sub_bk = lax.broadcasted_iota(jnp.int32, (_B * _K, 1), 0)
    exp_bk = jnp.where(
        (sub_bk // _K) == lax.broadcasted_iota(jnp.int32, (_B * _K, _B), 1), 1.0, 0.0)
    pair2 = jnp.concatenate(
        [gath, jnp.dot(exp_bk, phrs, preferred_element_type=f32)], axis=1)

    h2 = _leaky(jnp.dot(pair2, W1st_ref[...], preferred_element_type=f32)
                + b1st_ref[...])
    sim2 = jnp.dot(h2, W2st_ref[...], preferred_element_type=f32) + b2st_ref[...]

    h3 = _leaky(jnp.dot(pair2, W1rt_ref[...], preferred_element_type=f32)
                + b1rt_ref[...])
    reg_out[...] = jnp.dot(h3, W2rt_ref[...], preferred_element_type=f32) + b2rt_ref[...]

    fused = sim2 * scores_all
    for b in range(_B):
        det_b = jnp.dot(onehotsT[b], fused[b * _K:(b + 1) * _K, :],
                        preferred_element_type=f32)
        touched = jnp.dot(onehotsT[b], jnp.ones((_K, 1), f32),
                          preferred_element_type=f32)
        det_out[pl.ds(b * _N, _N), :] = jnp.where(touched > 0.5, det_b, _NEG)


@jax.jit
def kernel(box_features, phrase_embed, target_id,
           W1_sim, b1_sim, W2_sim, b2_sim,
           W1_reg, b1_reg, W2_reg, b2_reg,
           W1_sim_topN, b1_sim_topN, W2_sim_topN, b2_sim_topN,
           W1_reg_topN, b1_reg_topN, W2_reg_topN, b2_reg_topN):
    del W1_reg, b1_reg, W2_reg, b2_reg

    f32 = jnp.float32
    args = (
        box_features.reshape(_B * _N, _D_REC),
        phrase_embed.reshape(_B * _P, _D_PHR),
        W1_sim, b1_sim.reshape(1, _HID), W2_sim, b2_sim.reshape(1, 1),
        W1_sim_topN, b1_sim_topN.reshape(1, _HID), W2_sim_topN,
        b2_sim_topN.reshape(1, 1),
        W1_reg_topN, b1_reg_topN.reshape(1, _HID), W2_reg_topN,
        b2_reg_topN.reshape(1, 6),
    )

    def full(a):
        return pl.BlockSpec(a.shape, lambda i, tid: (0,) * a.ndim)

    grid_spec = pltpu.PrefetchScalarGridSpec(
        num_scalar_prefetch=1,
        grid=(1,),
        in_specs=[full(a) for a in args],
        out_specs=[
            pl.BlockSpec((_B * _N, 1), lambda i, tid: (0, 0)),
            pl.BlockSpec((_B * _N, 1), lambda i, tid: (0, 0)),
            pl.BlockSpec((_B * _K, 6), lambda i, tid: (0, 0)),
        ],
    )

    sim2d, det2d, reg2d = pl.pallas_call(
        _body,
        grid_spec=grid_spec,
        out_shape=[
            jax.ShapeDtypeStruct((_B * _N, 1), f32),
            jax.ShapeDtypeStruct((_B * _N, 1), f32),
            jax.ShapeDtypeStruct((_B * _K, 6), f32),
        ],
    )(target_id, *args)

    return (sim2d.reshape(_B, _N), det2d.reshape(_B, _N),
            reg2d.reshape(_B, _K, 6))


# speedup vs baseline: 5.9885x; 1.0930x over previous
import jax
import jax.numpy as jnp
from jax import lax
from jax.experimental import pallas as pl
from jax.experimental.pallas import tpu as pltpu

_B, _P, _N, _K = 4, 25, 256, 8
_D_REC, _D_PHR = 128, 768
_HID = 256
_NEG = -1e9


def _leaky(x):
    return jnp.where(x > 0, x, 0.01 * x)


def _body(tid_ref, box_ref, phr_ref,
          W1s_ref, b1s_ref, W2s_ref, b2s_ref,
          W1st_ref, b1st_ref, W2st_ref, b2st_ref,
          W1rt_ref, b1rt_ref, W2rt_ref, b2rt_ref,
          sim_out, det_out, reg_out):
    f32 = jnp.float32

    sub_b = lax.broadcasted_iota(jnp.int32, (_B, _B * _P), 0)
    lane_bp = lax.broadcasted_iota(jnp.int32, (_B, _B * _P), 1)
    sel = jnp.zeros((_B, _B * _P), f32)
    for b in range(_B):
        sel = jnp.where((sub_b == b) & (lane_bp == b * _P + tid_ref[b]), 1.0, sel)
    phrs = jnp.dot(sel, phr_ref[...], preferred_element_type=f32)

    box_all = box_ref[...]
    sub_bn = lax.broadcasted_iota(jnp.int32, (_B * _N, 1), 0)
    exp_bn = jnp.where(
        (sub_bn // _N) == lax.broadcasted_iota(jnp.int32, (_B * _N, _B), 1), 1.0, 0.0)
    pair = jnp.concatenate(
        [box_all, jnp.dot(exp_bn, phrs, preferred_element_type=f32)], axis=1)
    h = _leaky(jnp.dot(pair, W1s_ref[...], preferred_element_type=f32)
               + b1s_ref[...])
    sim_col = jnp.dot(h, W2s_ref[...], preferred_element_type=f32) + b2s_ref[...]
    sim_row = jnp.transpose(sim_col)
    sim_out[...] = sim_row

    lane_n = lax.broadcasted_iota(jnp.int32, (1, _N), 1)
    lane_k = lax.broadcasted_iota(jnp.int32, (1, _K), 1)
    sub_kn0 = lax.broadcasted_iota(jnp.int32, (_K, _N), 0)
    lane_kn1 = lax.broadcasted_iota(jnp.int32, (_K, _N), 1)

    score_rows = []
    onehots = []
    for b in range(_B):
        work = sim_row[:, b * _N:(b + 1) * _N]
        sc = jnp.zeros((1, _K), f32)
        oh = jnp.zeros((_K, _N), f32)
        for k in range(_K):
            m = jnp.max(work)
            idx = jnp.min(jnp.where(work == m, lane_n, _N))
            sc = jnp.where(lane_k == k, m, sc)
            oh = jnp.where((sub_kn0 == k) & (lane_kn1 == idx), 1.0, oh)
            work = jnp.where(lane_n == idx, -jnp.inf, work)
        score_rows.append(sc)
        onehots.append(oh)

    gath = jnp.concatenate(
        [jnp.dot(onehots[b], box_all[b * _N:(b + 1) * _N, :],
                 preferred_element_type=f32) for b in range(_B)], axis=0)

    sub_bk = lax.broadcasted_iota(jnp.int32, (_B * _K, 1), 0)
    exp_bk = jnp.where(
        (sub_bk // _K) == lax.broadcasted_iota(jnp.int32, (_B * _K, _B), 1), 1.0, 0.0)
    pair2 = jnp.concatenate(
        [gath, jnp.dot(exp_bk, phrs, preferred_element_type=f32)], axis=1)

    h2 = _leaky(jnp.dot(pair2, W1st_ref[...], preferred_element_type=f32)
                + b1st_ref[...])
    sim2 = jnp.dot(h2, W2st_ref[...], preferred_element_type=f32) + b2st_ref[...]

    h3 = _leaky(jnp.dot(pair2, W1rt_ref[...], preferred_element_type=f32)
                + b1rt_ref[...])
    reg_out[...] = jnp.dot(h3, W2rt_ref[...], preferred_element_type=f32) + b2rt_ref[...]

    sim2_row = jnp.transpose(sim2)
    fused = sim2_row * jnp.concatenate(score_rows, axis=1)
    ones_k = jnp.ones((1, _K), f32)
    det_parts = []
    for b in range(_B):
        det_b = jnp.dot(fused[:, b * _K:(b + 1) * _K], onehots[b],
                        preferred_element_type=f32)
        touched = jnp.dot(ones_k, onehots[b], preferred_element_type=f32)
        det_parts.append(jnp.where(touched > 0.5, det_b, _NEG))
    det_out[...] = jnp.concatenate(det_parts, axis=1)


@jax.jit
def kernel(box_features, phrase_embed, target_id,
           W1_sim, b1_sim, W2_sim, b2_sim,
           W1_reg, b1_reg, W2_reg, b2_reg,
           W1_sim_topN, b1_sim_topN, W2_sim_topN, b2_sim_topN,
           W1_reg_topN, b1_reg_topN, W2_reg_topN, b2_reg_topN):
    del W1_reg, b1_reg, W2_reg, b2_reg

    f32 = jnp.float32
    args = (
        box_features.reshape(_B * _N, _D_REC),
        phrase_embed.reshape(_B * _P, _D_PHR),
        W1_sim, b1_sim.reshape(1, _HID), W2_sim, b2_sim.reshape(1, 1),
        W1_sim_topN, b1_sim_topN.reshape(1, _HID), W2_sim_topN,
        b2_sim_topN.reshape(1, 1),
        W1_reg_topN, b1_reg_topN.reshape(1, _HID), W2_reg_topN,
        b2_reg_topN.reshape(1, 6),
    )

    def full(a):
        return pl.BlockSpec(a.shape, lambda i, tid: (0,) * a.ndim)

    grid_spec = pltpu.PrefetchScalarGridSpec(
        num_scalar_prefetch=1,
        grid=(1,),
        in_specs=[full(a) for a in args],
        out_specs=[
            pl.BlockSpec((1, _B * _N), lambda i, tid: (0, 0)),
            pl.BlockSpec((1, _B * _N), lambda i, tid: (0, 0)),
            pl.BlockSpec((_B * _K, 6), lambda i, tid: (0, 0)),
        ],
    )

    sim2d, det2d, reg2d = pl.pallas_call(
        _body,
        grid_spec=grid_spec,
        out_shape=[
            jax.ShapeDtypeStruct((1, _B * _N), f32),
            jax.ShapeDtypeStruct((1, _B * _N), f32),
            jax.ShapeDtypeStruct((_B * _K, 6), f32),
        ],
    )(target_id, *args)

    return (sim2d.reshape(_B, _N), det2d.reshape(_B, _N),
            reg2d.reshape(_B, _K, 6))


# speedup vs baseline: 8.3756x; 1.3986x over previous
import jax
import jax.numpy as jnp
from jax import lax
from jax.experimental import pallas as pl
from jax.experimental.pallas import tpu as pltpu

_B, _P, _N, _K = 4, 25, 256, 8
_D_REC, _D_PHR = 128, 768
_HID = 256
_NEG = -1e9


def _leaky(x):
    return jnp.where(x > 0, x, 0.01 * x)


def _body(tid_ref, box_ref, phr_ref,
          W1s_ref, b1s_ref, W2s_ref, b2s_ref,
          W1st_ref, b1st_ref, W2st_ref, b2st_ref,
          W1rt_ref, b1rt_ref, W2rt_ref, b2rt_ref,
          sim_out, det_out, reg_out):
    f32 = jnp.float32

    sub_b = lax.broadcasted_iota(jnp.int32, (_B, _B * _P), 0)
    lane_bp = lax.broadcasted_iota(jnp.int32, (_B, _B * _P), 1)
    sel = jnp.zeros((_B, _B * _P), f32)
    for b in range(_B):
        sel = jnp.where((sub_b == b) & (lane_bp == b * _P + tid_ref[b]), 1.0, sel)
    phrs = jnp.dot(sel, phr_ref[...], preferred_element_type=f32)

    box_all = box_ref[...]
    sub_bn = lax.broadcasted_iota(jnp.int32, (_B * _N, 1), 0)
    exp_bn = jnp.where(
        (sub_bn // _N) == lax.broadcasted_iota(jnp.int32, (_B * _N, _B), 1), 1.0, 0.0)
    pair = jnp.concatenate(
        [box_all, jnp.dot(exp_bn, phrs, preferred_element_type=f32)], axis=1)
    h = _leaky(jnp.dot(pair, W1s_ref[...], preferred_element_type=f32)
               + b1s_ref[...])
    sim_col = jnp.dot(h, W2s_ref[...], preferred_element_type=f32) + b2s_ref[...]
    sim_row = jnp.transpose(sim_col)
    sim_out[...] = sim_row

    work = jnp.concatenate(
        [sim_row[:, b * _N:(b + 1) * _N] for b in range(_B)], axis=0)
    lane_n = lax.broadcasted_iota(jnp.int32, (1, _N), 1)
    lane_k = lax.broadcasted_iota(jnp.int32, (1, _K), 1)
    scores = jnp.zeros((_B, _K), f32)
    ids = jnp.zeros((_B, _K), jnp.int32)
    for k in range(_K):
        m = jnp.max(work, axis=1, keepdims=True)
        idx = jnp.min(jnp.where(work == m, lane_n, _N), axis=1, keepdims=True)
        scores = jnp.where(lane_k == k, m, scores)
        ids = jnp.where(lane_k == k, idx, ids)
        work = jnp.where(lane_n == idx, -jnp.inf, work)

    sub_bk = lax.broadcasted_iota(jnp.int32, (_B * _K, 1), 0)
    exp_bk = jnp.where(
        (sub_bk // _K) == lax.broadcasted_iota(jnp.int32, (_B * _K, _B), 1), 1.0, 0.0)
    pickk = jnp.where(
        (sub_bk % _K) == lax.broadcasted_iota(jnp.int32, (_B * _K, _K), 1), 1.0, 0.0)
    ids_rows = jnp.dot(exp_bk, ids.astype(f32), preferred_element_type=f32)
    ids_col = jnp.sum(ids_rows * pickk, axis=1, keepdims=True)
    scores_rows = jnp.dot(exp_bk, scores, preferred_element_type=f32)
    scores_col = jnp.sum(scores_rows * pickk, axis=1, keepdims=True)

    colid = ids_col.astype(jnp.int32) + (sub_bk // _K) * _N
    lane_bn = lax.broadcasted_iota(jnp.int32, (_B * _K, _B * _N), 1)
    big_oh = jnp.where(lane_bn == colid, 1.0, 0.0)

    gath = jnp.dot(big_oh, box_all, preferred_element_type=f32)

    pair2 = jnp.concatenate(
        [gath, jnp.dot(exp_bk, phrs, preferred_element_type=f32)], axis=1)

    h2 = _leaky(jnp.dot(pair2, W1st_ref[...], preferred_element_type=f32)
                + b1st_ref[...])
    sim2 = jnp.dot(h2, W2st_ref[...], preferred_element_type=f32) + b2st_ref[...]

    h3 = _leaky(jnp.dot(pair2, W1rt_ref[...], preferred_element_type=f32)
                + b1rt_ref[...])
    reg_out[...] = jnp.dot(h3, W2rt_ref[...], preferred_element_type=f32) + b2rt_ref[...]

    fused_row = jnp.transpose(sim2 * scores_col)
    det_row = jnp.dot(fused_row, big_oh, preferred_element_type=f32)
    touched = jnp.dot(jnp.ones((1, _B * _K), f32), big_oh,
                      preferred_element_type=f32)
    det_out[...] = jnp.where(touched > 0.5, det_row, _NEG)


@jax.jit
def kernel(box_features, phrase_embed, target_id,
           W1_sim, b1_sim, W2_sim, b2_sim,
           W1_reg, b1_reg, W2_reg, b2_reg,
           W1_sim_topN, b1_sim_topN, W2_sim_topN, b2_sim_topN,
           W1_reg_topN, b1_reg_topN, W2_reg_topN, b2_reg_topN):
    del W1_reg, b1_reg, W2_reg, b2_reg

    f32 = jnp.float32
    args = (
        box_features.reshape(_B * _N, _D_REC),
        phrase_embed.reshape(_B * _P, _D_PHR),
        W1_sim, b1_sim.reshape(1, _HID), W2_sim, b2_sim.reshape(1, 1),
        W1_sim_topN, b1_sim_topN.reshape(1, _HID), W2_sim_topN,
        b2_sim_topN.reshape(1, 1),
        W1_reg_topN, b1_reg_topN.reshape(1, _HID), W2_reg_topN,
        b2_reg_topN.reshape(1, 6),
    )

    def full(a):
        return pl.BlockSpec(a.shape, lambda i, tid: (0,) * a.ndim)

    grid_spec = pltpu.PrefetchScalarGridSpec(
        num_scalar_prefetch=1,
        grid=(1,),
        in_specs=[full(a) for a in args],
        out_specs=[
            pl.BlockSpec((1, _B * _N), lambda i, tid: (0, 0)),
            pl.BlockSpec((1, _B * _N), lambda i, tid: (0, 0)),
            pl.BlockSpec((_B * _K, 6), lambda i, tid: (0, 0)),
        ],
    )

    sim2d, det2d, reg2d = pl.pallas_call(
        _body,
        grid_spec=grid_spec,
        out_shape=[
            jax.ShapeDtypeStruct((1, _B * _N), f32),
            jax.ShapeDtypeStruct((1, _B * _N), f32),
            jax.ShapeDtypeStruct((_B * _K, 6), f32),
        ],
    )(target_id, *args)

    return (sim2d.reshape(_B, _N), det2d.reshape(_B, _N),
            reg2d.reshape(_B, _K, 6))
